# baseline (speedup 1.0000x reference)
import functools

import jax
import jax.numpy as jnp
from jax import lax
from jax.experimental import pallas as pl
from jax.experimental.pallas import tpu as pltpu
from jax.experimental.pallas import tpu_sc as plsc

NC = 2
NS = 16
NW = NC * NS
L = 16
CHUNK = 128
SC_FRAC = 2
TC_BLK = 256


def _sc_gather(t_sc, table, b_sc, D):
    b_per_w = b_sc // NW
    n_chunks = b_per_w // CHUNK
    mesh = plsc.VectorSubcoreMesh(core_axis_name="c", subcore_axis_name="s")

    @functools.partial(
        pl.kernel,
        out_type=jax.ShapeDtypeStruct((b_sc, D), jnp.float32),
        mesh=mesh,
        scratch_types=[
            pltpu.VMEM((b_per_w,), jnp.float32),
            pltpu.VMEM((n_chunks, CHUNK), jnp.int32),
            pltpu.VMEM((b_per_w, D), jnp.float32),
            pltpu.SemaphoreType.DMA,
            pltpu.SemaphoreType.DMA,
        ],
        compiler_params=pltpu.CompilerParams(use_tc_tiling_on_sc=False),
    )
    def _emb(t_hbm, table_hbm, out_hbm, t_v, idx_v, rows_v, gsem, wsem):
        wid = lax.axis_index("s") * NC + lax.axis_index("c")
        base = wid * b_per_w

        pltpu.sync_copy(t_hbm.at[pl.ds(base, b_per_w)], t_v)

        gathers = []
        for j in range(n_chunks):
            for i in range(CHUNK // L):
                v = t_v[pl.ds(j * CHUNK + i * L, L)]
                idx_v[j, pl.ds(i * L, L)] = (v * 999.0).astype(jnp.int32)
            gathers.append(
                pltpu.async_copy(
                    table_hbm.at[idx_v.at[j]],
                    rows_v.at[pl.ds(j * CHUNK, CHUNK)],
                    gsem,
                )
            )
        writes = []
        for j in range(n_chunks):
            gathers[j].wait()
            writes.append(
                pltpu.async_copy(
                    rows_v.at[pl.ds(j * CHUNK, CHUNK)],
                    out_hbm.at[pl.ds(base + j * CHUNK, CHUNK)],
                    wsem,
                )
            )
        for w in writes:
            w.wait()

    return _emb(t_sc, table)


def _tc_onehot(t_tc, table, b_tc, V, D):
    n_blk = b_tc // TC_BLK

    def _body(t_ref, table_ref, out_ref):
        idx = (t_ref[0, 0, :] * 999.0).astype(jnp.int32)
        iota = lax.broadcasted_iota(jnp.int32, (TC_BLK, V), 1)
        oh = (iota == idx[:, None]).astype(jnp.float32)
        out_ref[...] = jnp.dot(
            oh, table_ref[...], preferred_element_type=jnp.float32
        )

    return pl.pallas_call(
        _body,
        grid=(n_blk,),
        in_specs=[
            pl.BlockSpec((1, 1, TC_BLK), lambda i: (i, 0, 0)),
            pl.BlockSpec((V, D), lambda i: (0, 0)),
        ],
        out_specs=pl.BlockSpec((TC_BLK, D), lambda i: (i, 0)),
        out_shape=jax.ShapeDtypeStruct((b_tc, D), jnp.float32),
    )(t_tc.reshape(n_blk, 1, TC_BLK), table)


@jax.jit
def kernel(t, table):
    B = t.shape[0]
    V, D = table.shape
    b_sc = B // SC_FRAC
    b_tc = B - b_sc

    out_sc = _sc_gather(t[:b_sc], table, b_sc, D)
    out_tc = _tc_onehot(t[b_sc:], table, b_tc, V, D)
    return jnp.concatenate([out_sc, out_tc], axis=0)


# speedup vs baseline: 1.0952x; 1.0952x over previous
import functools

import jax
import jax.numpy as jnp
from jax import lax
from jax.experimental import pallas as pl
from jax.experimental.pallas import tpu as pltpu
from jax.experimental.pallas import tpu_sc as plsc

NC = 2
NS = 16
NW = NC * NS
L = 16
CHUNK = 128
SC_NUM = 1
SC_DEN = 2
TC_BLK = 256


def _sc_gather(t, table, b_sc, B, D):
    b_per_w = b_sc // NW
    n_chunks = max(b_per_w // CHUNK, 1)
    chunk = b_per_w // n_chunks
    mesh = plsc.VectorSubcoreMesh(core_axis_name="c", subcore_axis_name="s")

    @functools.partial(
        pl.kernel,
        out_type=jax.ShapeDtypeStruct((B, D), jnp.float32),
        mesh=mesh,
        scratch_types=[
            pltpu.VMEM((b_per_w,), jnp.float32),
            pltpu.VMEM((n_chunks, chunk), jnp.int32),
            pltpu.VMEM((b_per_w, D), jnp.float32),
            pltpu.SemaphoreType.DMA,
            pltpu.SemaphoreType.DMA,
        ],
        compiler_params=pltpu.CompilerParams(use_tc_tiling_on_sc=False),
    )
    def _emb(t_hbm, table_hbm, out_hbm, t_v, idx_v, rows_v, gsem, wsem):
        wid = lax.axis_index("s") * NC + lax.axis_index("c")
        base = wid * b_per_w

        pltpu.sync_copy(t_hbm.at[pl.ds(base, b_per_w)], t_v)

        gathers = []
        for j in range(n_chunks):
            for i in range(chunk // L):
                v = t_v[pl.ds(j * chunk + i * L, L)]
                idx_v[j, pl.ds(i * L, L)] = (v * 999.0).astype(jnp.int32)
            gathers.append(
                pltpu.async_copy(
                    table_hbm.at[idx_v.at[j]],
                    rows_v.at[pl.ds(j * chunk, chunk)],
                    gsem,
                )
            )
        writes = []
        for j in range(n_chunks):
            gathers[j].wait()
            writes.append(
                pltpu.async_copy(
                    rows_v.at[pl.ds(j * chunk, chunk)],
                    out_hbm.at[pl.ds(base + j * chunk, chunk)],
                    wsem,
                )
            )
        for w in writes:
            w.wait()

    return _emb(t, table)


def _tc_onehot(t2, table, off_blk, b_tc, V, D):
    n_blk = b_tc // TC_BLK

    def _body(t_ref, table_ref, out_ref):
        idx = (t_ref[0, :] * 999.0).astype(jnp.int32)
        iota = lax.broadcasted_iota(jnp.int32, (TC_BLK, V), 1)
        oh = (iota == idx[:, None]).astype(jnp.float32)
        out_ref[...] = jnp.dot(
            oh, table_ref[...], preferred_element_type=jnp.float32
        )

    return pl.pallas_call(
        _body,
        grid=(n_blk,),
        in_specs=[
            pl.BlockSpec((1, TC_BLK), lambda i: (0, i + off_blk)),
            pl.BlockSpec((V, D), lambda i: (0, 0)),
        ],
        out_specs=pl.BlockSpec((TC_BLK, D), lambda i: (i, 0)),
        out_shape=jax.ShapeDtypeStruct((b_tc, D), jnp.float32),
    )(t2, table)


@jax.jit
def kernel(t, table):
    B = t.shape[0]
    V, D = table.shape
    b_sc = B * SC_NUM // SC_DEN
    b_tc = B - b_sc

    out = _sc_gather(t, table, b_sc, B, D)
    out_tc = _tc_onehot(t.reshape(1, B), table, b_sc // TC_BLK, b_tc, V, D)
    return lax.dynamic_update_slice(out, out_tc, (b_sc, 0))


# speedup vs baseline: 1.2042x; 1.0995x over previous
import functools

import jax
import jax.numpy as jnp
from jax import lax
from jax.experimental import pallas as pl
from jax.experimental.pallas import tpu as pltpu
from jax.experimental.pallas import tpu_sc as plsc

NC = 2
NS = 16
NW = NC * NS
L = 16
CHUNK = 128
SC_NUM = 1
SC_DEN = 2
TC_BLK = 256


def _sc_gather(t, table, b_sc, B, D):
    b_per_w = b_sc // NW
    n_chunks = max(b_per_w // CHUNK, 1)
    chunk = b_per_w // n_chunks
    mesh = plsc.VectorSubcoreMesh(core_axis_name="c", subcore_axis_name="s")

    @functools.partial(
        pl.kernel,
        out_type=jax.ShapeDtypeStruct((b_sc, D), jnp.float32),
        mesh=mesh,
        scratch_types=[
            pltpu.VMEM((b_per_w,), jnp.float32),
            pltpu.VMEM((n_chunks, chunk), jnp.int32),
            pltpu.VMEM((b_per_w, D), jnp.float32),
            pltpu.SemaphoreType.DMA,
            pltpu.SemaphoreType.DMA,
        ],
        compiler_params=pltpu.CompilerParams(use_tc_tiling_on_sc=False),
    )
    def _emb(t_hbm, table_hbm, out_hbm, t_v, idx_v, rows_v, gsem, wsem):
        wid = lax.axis_index("s") * NC + lax.axis_index("c")
        base = wid * b_per_w

        pltpu.sync_copy(t_hbm.at[pl.ds(base, b_per_w)], t_v)

        gathers = []
        for j in range(n_chunks):
            for i in range(chunk // L):
                v = t_v[pl.ds(j * chunk + i * L, L)]
                idx_v[j, pl.ds(i * L, L)] = (v * 999.0).astype(jnp.int32)
            gathers.append(
                pltpu.async_copy(
                    table_hbm.at[idx_v.at[j]],
                    rows_v.at[pl.ds(j * chunk, chunk)],
                    gsem,
                )
            )
        writes = []
        for j in range(n_chunks):
            gathers[j].wait()
            writes.append(
                pltpu.async_copy(
                    rows_v.at[pl.ds(j * chunk, chunk)],
                    out_hbm.at[pl.ds(base + j * chunk, chunk)],
                    wsem,
                )
            )
        for w in writes:
            w.wait()

    return _emb(t, table)


def _tc_onehot(t, table, off_blk, b_tc, B, V, D):
    n_blk = b_tc // TC_BLK

    def _body(t_ref, table_ref, out_ref):
        idx = (t_ref[...] * 999.0).astype(jnp.int32)
        iota = lax.broadcasted_iota(jnp.int32, (TC_BLK, V), 1)
        oh = (iota == idx[:, None]).astype(jnp.float32)
        out_ref[...] = jnp.dot(
            oh, table_ref[...], preferred_element_type=jnp.float32
        )

    return pl.pallas_call(
        _body,
        grid=(n_blk,),
        in_specs=[
            pl.BlockSpec((TC_BLK,), lambda i: (i + off_blk,)),
            pl.BlockSpec((V, D), lambda i: (0, 0)),
        ],
        out_specs=pl.BlockSpec((TC_BLK, D), lambda i: (i + off_blk, 0)),
        out_shape=jax.ShapeDtypeStruct((B, D), jnp.float32),
    )(t, table)


@jax.jit
def kernel(t, table):
    B = t.shape[0]
    V, D = table.shape
    b_sc = B * SC_NUM // SC_DEN
    b_tc = B - b_sc

    out_sc = _sc_gather(t, table, b_sc, B, D)
    out = _tc_onehot(t, table, b_sc // TC_BLK, b_tc, B, V, D)
    return lax.dynamic_update_slice(out, out_sc, (0, 0))
